# baseline (speedup 1.0000x reference)
import functools

import jax
import jax.numpy as jnp
from jax import lax
from jax.experimental import pallas as pl
from jax.experimental.pallas import tpu as pltpu
from jax.experimental.pallas import tpu_sc as plsc

NHEAD = 8
DIM = 32
TOPK = 16
HC = 32
LC = HC * HC
HF = 64
LF = HF * HF
CDIM = 4 * DIM
SCALE_F = 1.0 / (DIM ** 0.5)

ROWBLK = 256
CELLBLK = 128

NGRP = 1
GH = NHEAD // NGRP
NWORK = 32
NIDX = GH * LC * TOPK
ROWS_PER_W = NIDX // NWORK
CHUNK = 128
NCHUNK = ROWS_PER_W // CHUNK


def _coarse_body(q_ref, k_ref, v_ref, msg_ref, idx_ref):
    h = pl.program_id(0)
    qd = q_ref[0]
    kd = k_ref[0]
    vd = v_ref[0]
    s = lax.dot_general(qd, kd, (((0,), (0,)), ((), ())),
                        preferred_element_type=jnp.float32) * SCALE_F
    m = jnp.max(s, axis=1, keepdims=True)
    e = jnp.exp(s - m)
    z = jnp.sum(e, axis=1, keepdims=True)
    msg_ref[0] = lax.dot_general(e, vd, (((1,), (1,)), ((), ())),
                                 preferred_element_type=jnp.float32) / z

    iota = lax.broadcasted_iota(jnp.int32, (ROWBLK, LC), 1).astype(jnp.float32)
    w = e
    cols = []
    for _ in range(TOPK):
        mj = jnp.max(w, axis=1, keepdims=True)
        ij = jnp.min(jnp.where(w == mj, iota, float(LC)), axis=1,
                     keepdims=True)
        cols.append(ij)
        w = jnp.where(iota == ij, -1.0, w)
    idx = jnp.concatenate(cols, axis=1).astype(jnp.int32)
    idx_ref[0] = idx + h * LC


def _coarse_call(qd, kd, vd):
    grid = (GH, LC // ROWBLK)
    return pl.pallas_call(
        _coarse_body,
        grid=grid,
        in_specs=[
            pl.BlockSpec((1, DIM, ROWBLK), lambda h, r: (h, 0, r)),
            pl.BlockSpec((1, DIM, LC), lambda h, r: (h, 0, 0)),
            pl.BlockSpec((1, DIM, LC), lambda h, r: (h, 0, 0)),
        ],
        out_specs=[
            pl.BlockSpec((1, ROWBLK, DIM), lambda h, r: (h, r, 0)),
            pl.BlockSpec((1, ROWBLK, TOPK), lambda h, r: (h, r, 0)),
        ],
        out_shape=[
            jax.ShapeDtypeStruct((GH, LC, DIM), jnp.float32),
            jax.ShapeDtypeStruct((GH, LC, TOPK), jnp.int32),
        ],
    )(qd, kd, vd)


def _sc_gather_body(tk, tv, idx_hbm, outk, outv,
                    idx0, idx1, kb0, kb1, vb0, vb1,
                    semg0, semg1, semw0, semw1):
    wid = lax.axis_index("s") * 2 + lax.axis_index("c")
    base = wid * ROWS_PER_W
    idxs = (idx0, idx1)
    kbs = (kb0, kb1)
    vbs = (vb0, vb1)
    semg = (semg0, semg1)
    semw = (semw0, semw1)

    pltpu.sync_copy(idx_hbm.at[pl.ds(base, CHUNK)], idx0)
    pltpu.async_copy(tk.at[idx0], kb0, semg0)
    pltpu.async_copy(tv.at[idx0], vb0, semg0)

    def pair(i2, carry):
        for b in range(2):
            i = i2 * 2 + b
            nb = 1 - b

            @pl.when(i > 0)
            def _():
                pltpu.make_async_copy(kbs[nb], outk.at[pl.ds(0, CHUNK)],
                                      semw[nb]).wait()
                pltpu.make_async_copy(vbs[nb], outv.at[pl.ds(0, CHUNK)],
                                      semw[nb]).wait()

            @pl.when(i + 1 < NCHUNK)
            def _():
                off1 = base + (i + 1) * CHUNK
                pltpu.sync_copy(idx_hbm.at[pl.ds(off1, CHUNK)], idxs[nb])
                pltpu.async_copy(tk.at[idxs[nb]], kbs[nb], semg[nb])
                pltpu.async_copy(tv.at[idxs[nb]], vbs[nb], semg[nb])

            pltpu.make_async_copy(tk.at[idxs[b]], kbs[b], semg[b]).wait()
            pltpu.make_async_copy(tv.at[idxs[b]], vbs[b], semg[b]).wait()
            off = base + i * CHUNK
            pltpu.async_copy(kbs[b], outk.at[pl.ds(off, CHUNK)], semw[b])
            pltpu.async_copy(vbs[b], outv.at[pl.ds(off, CHUNK)], semw[b])
        return carry

    lax.fori_loop(0, NCHUNK // 2, pair, 0)

    pltpu.make_async_copy(kb1, outk.at[pl.ds(0, CHUNK)], semw1).wait()
    pltpu.make_async_copy(vb1, outv.at[pl.ds(0, CHUNK)], semw1).wait()


def _sc_gather_call(table_k, table_v, gidx):
    mesh = plsc.VectorSubcoreMesh(core_axis_name="c", subcore_axis_name="s")
    kfn = functools.partial(
        pl.kernel,
        mesh=mesh,
        out_type=[
            jax.ShapeDtypeStruct((NIDX, CDIM), jnp.float32),
            jax.ShapeDtypeStruct((NIDX, CDIM), jnp.float32),
        ],
        scratch_types=[
            pltpu.VMEM((CHUNK,), jnp.int32),
            pltpu.VMEM((CHUNK,), jnp.int32),
            pltpu.VMEM((CHUNK, CDIM), jnp.float32),
            pltpu.VMEM((CHUNK, CDIM), jnp.float32),
            pltpu.VMEM((CHUNK, CDIM), jnp.float32),
            pltpu.VMEM((CHUNK, CDIM), jnp.float32),
            pltpu.SemaphoreType.DMA,
            pltpu.SemaphoreType.DMA,
            pltpu.SemaphoreType.DMA,
            pltpu.SemaphoreType.DMA,
        ],
    )(_sc_gather_body)
    return kfn(table_k, table_v, gidx)


def _fine_body(q_ref, k_ref, v_ref, m1_ref, w_ref, out_ref):
    h = pl.program_id(1)
    kg = k_ref[...].reshape(CELLBLK, TOPK, CDIM)
    vg = v_ref[...].reshape(CELLBLK, TOPK, CDIM)
    m1 = m1_ref[0]
    w0 = w_ref[0, 0]
    w1 = w_ref[0, 1]
    base = w0 * m1

    grp = lax.broadcasted_iota(jnp.int32, (CDIM, 4), 0) // DIM
    gcol = lax.broadcasted_iota(jnp.int32, (CDIM, 4), 1)
    selg = (grp == gcol).astype(jnp.float32)
    lane = lax.broadcasted_iota(jnp.int32, (CDIM, DIM), 0) % DIM
    dcol = lax.broadcasted_iota(jnp.int32, (CDIM, DIM), 1)
    seld = (lane == dcol).astype(jnp.float32)
    seldT = seld.T
    selgT = selg.T
    rcell = lax.broadcasted_iota(jnp.int32, (CELLBLK * TOPK, CELLBLK), 0)
    ccol = lax.broadcasted_iota(jnp.int32, (CELLBLK * TOPK, CELLBLK), 1)
    asum = (rcell // TOPK == ccol).astype(jnp.float32)

    for t in range(4):
        qt = q_ref[0, t] * SCALE_F
        qrep = jnp.dot(qt, seldT, preferred_element_type=jnp.float32)
        prod = (kg * qrep[:, None, :]).reshape(CELLBLK * TOPK, CDIM)
        st = jnp.dot(jnp.dot(prod, selg, preferred_element_type=jnp.float32),
                     selgT, preferred_element_type=jnp.float32)
        st = st.reshape(CELLBLK, TOPK, CDIM)
        mt = jnp.max(jnp.max(st, axis=2, keepdims=True), axis=1, keepdims=True)
        et = jnp.exp(st - mt).reshape(CELLBLK * TOPK, CDIM)
        ev = et * vg.reshape(CELLBLK * TOPK, CDIM)
        mm = lax.dot_general(asum, ev, (((0,), (0,)), ((), ())),
                             preferred_element_type=jnp.float32)
        z128 = lax.dot_general(asum, et, (((0,), (0,)), ((), ())),
                               preferred_element_type=jnp.float32)
        msg = jnp.dot(mm, seld, preferred_element_type=jnp.float32) \
            / jnp.dot(z128, seld, preferred_element_type=jnp.float32)
        x, y = divmod(t, 2)
        out_ref[:, x, :, y, h, :] = (base + w1 * msg).reshape(4, HC, DIM)


def _fine_call(qg_t, outk, outv, msg1, wgt):
    nblk = LC // CELLBLK
    grid = (nblk, GH)
    return pl.pallas_call(
        _fine_body,
        grid=grid,
        in_specs=[
            pl.BlockSpec((1, 4, CELLBLK, DIM), lambda c, h: (h, 0, c, 0)),
            pl.BlockSpec((CELLBLK * TOPK, CDIM), lambda c, h: (h * nblk + c, 0)),
            pl.BlockSpec((CELLBLK * TOPK, CDIM), lambda c, h: (h * nblk + c, 0)),
            pl.BlockSpec((1, CELLBLK, DIM), lambda c, h: (h, c, 0)),
            pl.BlockSpec(memory_space=pltpu.SMEM),
        ],
        out_specs=pl.BlockSpec((4, 2, HC, 2, GH, DIM),
                               lambda c, h: (c, 0, 0, 0, 0, 0)),
        out_shape=jax.ShapeDtypeStruct((HC, 2, HC, 2, GH, DIM), jnp.float32),
    )(qg_t, outk, outv, msg1, wgt)


def _cell_rows(x):
    x = x.reshape(GH, DIM, HC, 2, HC, 2)
    x = jnp.transpose(x, (0, 2, 4, 3, 5, 1))
    return x.reshape(GH * LC, CDIM)


def kernel(q0, q1, k0, k1, v0, v1, weight):
    qd = q1.reshape(NHEAD, DIM, LC)
    kd = k1.reshape(NHEAD, DIM, LC)
    vd = v1.reshape(NHEAD, DIM, LC)
    q0r = q0.reshape(NHEAD, DIM, HC, 2, HC, 2)
    k0r = k0.reshape(NHEAD * DIM, HF, HF)
    v0r = v0.reshape(NHEAD * DIM, HF, HF)
    wgt = jax.nn.softmax(weight).reshape(1, 2)

    coarse = [
        _coarse_call(qd[g * GH:(g + 1) * GH],
                     kd[g * GH:(g + 1) * GH],
                     vd[g * GH:(g + 1) * GH])
        for g in range(NGRP)
    ]
    gath = [
        _sc_gather_call(_cell_rows(k0r[g * GH * DIM:(g + 1) * GH * DIM]),
                        _cell_rows(v0r[g * GH * DIM:(g + 1) * GH * DIM]),
                        coarse[g][1].reshape(-1))
        for g in range(NGRP)
    ]
    outs = []
    for g in range(NGRP):
        q0g = q0r[g * GH:(g + 1) * GH]
        qg_t = jnp.transpose(q0g, (0, 3, 5, 2, 4, 1)).reshape(GH, 4, LC, DIM)
        outk, outv = gath[g]
        outs.append(_fine_call(qg_t, outk, outv, coarse[g][0], wgt))

    out = outs[0] if NGRP == 1 else jnp.concatenate(outs, axis=4)
    return out.reshape(1, LF, NHEAD, DIM)


# speedup vs baseline: 1.0525x; 1.0525x over previous
import functools

import jax
import jax.numpy as jnp
from jax import lax
from jax.experimental import pallas as pl
from jax.experimental.pallas import tpu as pltpu
from jax.experimental.pallas import tpu_sc as plsc

NHEAD = 8
DIM = 32
TOPK = 16
HC = 32
LC = HC * HC
HF = 64
LF = HF * HF
CDIM = 4 * DIM
SCALE_F = 1.0 / (DIM ** 0.5)

ROWBLK = 256
CELLBLK = 128

NGRP = 1
GH = NHEAD // NGRP
NWORK = 32
NIDX = GH * LC * TOPK
ROWS_PER_W = NIDX // NWORK
CHUNK = 128
NCHUNK = ROWS_PER_W // CHUNK


def _coarse_body(q_ref, k_ref, v_ref, msg_ref, idx_ref):
    h = pl.program_id(0)
    qd = q_ref[0]
    kd = k_ref[0]
    vd = v_ref[0]
    s = lax.dot_general(qd, kd, (((0,), (0,)), ((), ())),
                        preferred_element_type=jnp.float32) * SCALE_F
    m = jnp.max(s, axis=1, keepdims=True)
    e = jnp.exp(s - m)
    z = jnp.sum(e, axis=1, keepdims=True)
    msg_ref[0] = lax.dot_general(e, vd, (((1,), (1,)), ((), ())),
                                 preferred_element_type=jnp.float32) / z

    iota = lax.broadcasted_iota(jnp.int32, (ROWBLK, LC), 1).astype(jnp.float32)
    w = e
    cols = []
    for _ in range(TOPK):
        mj = jnp.max(w, axis=1, keepdims=True)
        ij = jnp.min(jnp.where(w == mj, iota, float(LC)), axis=1,
                     keepdims=True)
        cols.append(ij)
        w = jnp.where(iota == ij, -1.0, w)
    idx = jnp.concatenate(cols, axis=1).astype(jnp.int32)
    idx_ref[0] = idx + h * LC


def _coarse_call(qd, kd, vd):
    grid = (GH, LC // ROWBLK)
    return pl.pallas_call(
        _coarse_body,
        grid=grid,
        in_specs=[
            pl.BlockSpec((1, DIM, ROWBLK), lambda h, r: (h, 0, r)),
            pl.BlockSpec((1, DIM, LC), lambda h, r: (h, 0, 0)),
            pl.BlockSpec((1, DIM, LC), lambda h, r: (h, 0, 0)),
        ],
        out_specs=[
            pl.BlockSpec((1, ROWBLK, DIM), lambda h, r: (h, r, 0)),
            pl.BlockSpec((1, ROWBLK, TOPK), lambda h, r: (h, r, 0)),
        ],
        out_shape=[
            jax.ShapeDtypeStruct((GH, LC, DIM), jnp.float32),
            jax.ShapeDtypeStruct((GH, LC, TOPK), jnp.int32),
        ],
    )(qd, kd, vd)


def _sc_gather_body(tk, tv, idx_hbm, outk, outv,
                    idx0, idx1, kb0, kb1, vb0, vb1,
                    semg0, semg1, semw0, semw1):
    wid = lax.axis_index("s") * 2 + lax.axis_index("c")
    base = wid * ROWS_PER_W
    idxs = (idx0, idx1)
    kbs = (kb0, kb1)
    vbs = (vb0, vb1)
    semg = (semg0, semg1)
    semw = (semw0, semw1)

    pltpu.sync_copy(idx_hbm.at[pl.ds(base, CHUNK)], idx0)
    pltpu.async_copy(tk.at[idx0], kb0, semg0)
    pltpu.async_copy(tv.at[idx0], vb0, semg0)

    def pair(i2, carry):
        for b in range(2):
            i = i2 * 2 + b
            nb = 1 - b

            @pl.when(i > 0)
            def _():
                pltpu.make_async_copy(kbs[nb], outk.at[pl.ds(0, CHUNK)],
                                      semw[nb]).wait()
                pltpu.make_async_copy(vbs[nb], outv.at[pl.ds(0, CHUNK)],
                                      semw[nb]).wait()

            @pl.when(i + 1 < NCHUNK)
            def _():
                off1 = base + (i + 1) * CHUNK
                pltpu.sync_copy(idx_hbm.at[pl.ds(off1, CHUNK)], idxs[nb])
                pltpu.async_copy(tk.at[idxs[nb]], kbs[nb], semg[nb])
                pltpu.async_copy(tv.at[idxs[nb]], vbs[nb], semg[nb])

            pltpu.make_async_copy(tk.at[idxs[b]], kbs[b], semg[b]).wait()
            pltpu.make_async_copy(tv.at[idxs[b]], vbs[b], semg[b]).wait()
            off = base + i * CHUNK
            pltpu.async_copy(kbs[b], outk.at[pl.ds(off, CHUNK)], semw[b])
            pltpu.async_copy(vbs[b], outv.at[pl.ds(off, CHUNK)], semw[b])
        return carry

    lax.fori_loop(0, NCHUNK // 2, pair, 0)

    pltpu.make_async_copy(kb1, outk.at[pl.ds(0, CHUNK)], semw1).wait()
    pltpu.make_async_copy(vb1, outv.at[pl.ds(0, CHUNK)], semw1).wait()


def _sc_gather_call(table_k, table_v, gidx):
    mesh = plsc.VectorSubcoreMesh(core_axis_name="c", subcore_axis_name="s")
    kfn = functools.partial(
        pl.kernel,
        mesh=mesh,
        out_type=[
            jax.ShapeDtypeStruct((NIDX, CDIM), jnp.float32),
            jax.ShapeDtypeStruct((NIDX, CDIM), jnp.float32),
        ],
        scratch_types=[
            pltpu.VMEM((CHUNK,), jnp.int32),
            pltpu.VMEM((CHUNK,), jnp.int32),
            pltpu.VMEM((CHUNK, CDIM), jnp.float32),
            pltpu.VMEM((CHUNK, CDIM), jnp.float32),
            pltpu.VMEM((CHUNK, CDIM), jnp.float32),
            pltpu.VMEM((CHUNK, CDIM), jnp.float32),
            pltpu.SemaphoreType.DMA,
            pltpu.SemaphoreType.DMA,
            pltpu.SemaphoreType.DMA,
            pltpu.SemaphoreType.DMA,
        ],
    )(_sc_gather_body)
    return kfn(table_k, table_v, gidx)


def _fine_body(q_ref, k_ref, v_ref, m1_ref, w_ref, out_ref):
    h = pl.program_id(1)
    kg = k_ref[...].reshape(CELLBLK, TOPK, CDIM)
    vg = v_ref[...].reshape(CELLBLK, TOPK, CDIM)
    m1 = m1_ref[0]
    w0 = w_ref[0, 0]
    w1 = w_ref[0, 1]
    base = w0 * m1

    grp = lax.broadcasted_iota(jnp.int32, (CDIM, 4), 0) // DIM
    gcol = lax.broadcasted_iota(jnp.int32, (CDIM, 4), 1)
    selg = (grp == gcol).astype(jnp.float32)
    lane = lax.broadcasted_iota(jnp.int32, (CDIM, DIM), 0) % DIM
    dcol = lax.broadcasted_iota(jnp.int32, (CDIM, DIM), 1)
    seld = (lane == dcol).astype(jnp.float32)
    seldT = seld.T
    selgT = selg.T

    for t in range(4):
        qt = q_ref[0, t] * SCALE_F
        qrep = jnp.dot(qt, seldT, preferred_element_type=jnp.float32)
        prod = (kg * qrep[:, None, :]).reshape(CELLBLK * TOPK, CDIM)
        st = jnp.dot(jnp.dot(prod, selg, preferred_element_type=jnp.float32),
                     selgT, preferred_element_type=jnp.float32)
        st = st.reshape(CELLBLK, TOPK, CDIM)
        mt = jnp.max(jnp.max(st, axis=2, keepdims=True), axis=1, keepdims=True)
        et = jnp.exp(st - mt)
        zt = jnp.sum(jnp.sum(et, axis=2, keepdims=True), axis=1, keepdims=True)
        mm = jnp.sum(et * vg, axis=1)
        msg = jnp.dot(mm, seld, preferred_element_type=jnp.float32) \
            * (DIM / zt[:, 0])
        x, y = divmod(t, 2)
        out_ref[:, x, :, y, h, :] = (base + w1 * msg).reshape(4, HC, DIM)


def _fine_call(qg_t, outk, outv, msg1, wgt):
    nblk = LC // CELLBLK
    grid = (nblk, GH)
    return pl.pallas_call(
        _fine_body,
        grid=grid,
        in_specs=[
            pl.BlockSpec((1, 4, CELLBLK, DIM), lambda c, h: (h, 0, c, 0)),
            pl.BlockSpec((CELLBLK * TOPK, CDIM), lambda c, h: (h * nblk + c, 0)),
            pl.BlockSpec((CELLBLK * TOPK, CDIM), lambda c, h: (h * nblk + c, 0)),
            pl.BlockSpec((1, CELLBLK, DIM), lambda c, h: (h, c, 0)),
            pl.BlockSpec(memory_space=pltpu.SMEM),
        ],
        out_specs=pl.BlockSpec((4, 2, HC, 2, GH, DIM),
                               lambda c, h: (c, 0, 0, 0, 0, 0)),
        out_shape=jax.ShapeDtypeStruct((HC, 2, HC, 2, GH, DIM), jnp.float32),
    )(qg_t, outk, outv, msg1, wgt)


def _cell_rows(x):
    x = x.reshape(GH, DIM, HC, 2, HC, 2)
    x = jnp.transpose(x, (0, 2, 4, 3, 5, 1))
    return x.reshape(GH * LC, CDIM)


def kernel(q0, q1, k0, k1, v0, v1, weight):
    qd = q1.reshape(NHEAD, DIM, LC)
    kd = k1.reshape(NHEAD, DIM, LC)
    vd = v1.reshape(NHEAD, DIM, LC)
    q0r = q0.reshape(NHEAD, DIM, HC, 2, HC, 2)
    k0r = k0.reshape(NHEAD * DIM, HF, HF)
    v0r = v0.reshape(NHEAD * DIM, HF, HF)
    wgt = jax.nn.softmax(weight).reshape(1, 2)

    coarse = [
        _coarse_call(qd[g * GH:(g + 1) * GH],
                     kd[g * GH:(g + 1) * GH],
                     vd[g * GH:(g + 1) * GH])
        for g in range(NGRP)
    ]
    gath = [
        _sc_gather_call(_cell_rows(k0r[g * GH * DIM:(g + 1) * GH * DIM]),
                        _cell_rows(v0r[g * GH * DIM:(g + 1) * GH * DIM]),
                        coarse[g][1].reshape(-1))
        for g in range(NGRP)
    ]
    outs = []
    for g in range(NGRP):
        q0g = q0r[g * GH:(g + 1) * GH]
        qg_t = jnp.transpose(q0g, (0, 3, 5, 2, 4, 1)).reshape(GH, 4, LC, DIM)
        outk, outv = gath[g]
        outs.append(_fine_call(qg_t, outk, outv, coarse[g][0], wgt))

    out = outs[0] if NGRP == 1 else jnp.concatenate(outs, axis=4)
    return out.reshape(1, LF, NHEAD, DIM)
